# Initial kernel scaffold; baseline (speedup 1.0000x reference)
import jax
import jax.numpy as jnp
from jax.experimental import pallas as pl


def kernel(obj_feats, phr_feats, pair_idxs, ws_w, ws_b, wo_w, wo_b, w_w, w_b, conv_w, conv_b, ln1_g, ln1_b, ln2_g, ln2_b, trans1_w, trans1_b, trans2_w, trans2_b):
    raise NotImplementedError("write your pallas kernel here")


# speedup vs baseline: 8.1243x; 8.1243x over previous
import jax
import jax.numpy as jnp
from jax.experimental import pallas as pl
from jax.experimental.pallas import tpu as pltpu

_N = 256
_D = 512
_E = 16384
_EB = 2048
_F32 = jnp.float32
_BF16 = jnp.bfloat16
_HI = jax.lax.Precision.HIGHEST


def _prep_body(obj_ref, ws_w_ref, ws_b_ref, wo_w_ref, wo_b_ref, w_w_ref,
               conv_w_ref, conv_b_ref, ln1_g_ref, ln1_b_ref,
               s_ref, o_ref, conv_ref, omega_ref):
    obj = obj_ref[...]
    s = jnp.dot(obj, ws_w_ref[...], preferred_element_type=_F32, precision=_HI) + ws_b_ref[...]
    s_mod = s * w_w_ref[...]
    o = jnp.dot(obj, wo_w_ref[...], preferred_element_type=_F32, precision=_HI) + wo_b_ref[...]
    s_ref[...] = s_mod.astype(_BF16)
    o_ref[...] = o.astype(_BF16)

    g = jax.lax.dot_general(obj, obj, (((1,), (1,)), ((), ())),
                            preferred_element_type=_F32, precision=_HI)
    rows = jax.lax.broadcasted_iota(jnp.int32, (_N, _N), 0)
    cols = jax.lax.broadcasted_iota(jnp.int32, (_N, _N), 1)
    eye = (rows == cols).astype(_F32)
    diag_col = jnp.sum(g * eye, axis=1, keepdims=True)
    diag_row = jnp.sum(g * eye, axis=0, keepdims=True)
    n2 = jnp.maximum(diag_col + diag_row - 2.0 * g, 0.0)
    omega = jnp.where(n2 < 0.25, 4.0, 0.0)
    omega = jnp.where((n2 >= 0.25) & (n2 < 1.0),
                      1.0 / jnp.maximum(n2, 1e-10), omega)
    omega = jnp.where(rows == cols, 0.0, omega)
    omega_ref[...] = omega

    mu = jnp.mean(obj, axis=1, keepdims=True)
    xc = obj - mu
    var = jnp.mean(xc * xc, axis=1, keepdims=True)
    xn = xc / jnp.sqrt(var + 1e-5) * ln1_g_ref[...] + ln1_b_ref[...]
    conv_ref[...] = jax.nn.relu(
        jnp.dot(xn, conv_w_ref[...], preferred_element_type=_F32, precision=_HI)
        + conv_b_ref[...])


def _edge_body(src_ref, dst_ref, phr_ref, s_ref, o_ref, wb_ref, atten_ref):
    src = src_ref[0, 0, :]
    dst = dst_ref[0, 0, :]
    ids = jax.lax.broadcasted_iota(jnp.int32, (_EB, _N), 1)
    oh_s = (src[:, None] == ids).astype(_BF16)
    oh_d = (dst[:, None] == ids).astype(_BF16)
    gs = jnp.dot(oh_s, s_ref[...], preferred_element_type=_F32)
    go = jnp.dot(oh_d, o_ref[...], preferred_element_type=_F32)
    t = gs * go * phr_ref[...]
    af = jnp.sum(t, axis=1) + wb_ref[0, 0]
    weighted = oh_s * af[:, None].astype(_BF16)
    contrib = jax.lax.dot_general(weighted, oh_d, (((0,), (0,)), ((), ())),
                                  preferred_element_type=_F32)

    @pl.when(pl.program_id(0) == 0)
    def _():
        atten_ref[...] = jnp.zeros_like(atten_ref)

    atten_ref[...] += contrib


def _finish_body(atten_ref, omega_ref, conv_ref, obj_ref, ln2_g_ref, ln2_b_ref,
                 t1w_ref, t1b_ref, t2w_ref, t2b_ref, out_ref):
    rows = jax.lax.broadcasted_iota(jnp.int32, (_N, _N), 0)
    cols = jax.lax.broadcasted_iota(jnp.int32, (_N, _N), 1)
    a = atten_ref[...] - jnp.where(rows == cols, 10000.0, 0.0)
    m = jnp.max(a, axis=1, keepdims=True)
    ex = jnp.exp(a - m)
    sm = ex / jnp.sum(ex, axis=1, keepdims=True)
    am = omega_ref[...] * sm
    context = jnp.dot(am, conv_ref[...], preferred_element_type=_F32, precision=_HI)
    outputs = obj_ref[...] + context
    mu = jnp.mean(outputs, axis=1, keepdims=True)
    xc = outputs - mu
    var = jnp.mean(xc * xc, axis=1, keepdims=True)
    xn = xc / jnp.sqrt(var + 1e-5) * ln2_g_ref[...] + ln2_b_ref[...]
    h = jax.nn.relu(jnp.dot(xn, t1w_ref[...], preferred_element_type=_F32, precision=_HI)
                    + t1b_ref[...])
    trans = jnp.dot(h, t2w_ref[...], preferred_element_type=_F32, precision=_HI) + t2b_ref[...]
    out_ref[...] = jax.nn.relu(outputs + trans)


def _full(shape, dtype=_F32):
    return pl.BlockSpec(shape, lambda *_: tuple(0 for _ in shape))


def kernel(obj_feats, phr_feats, pair_idxs, ws_w, ws_b, wo_w, wo_b, w_w, w_b,
           conv_w, conv_b, ln1_g, ln1_b, ln2_g, ln2_b,
           trans1_w, trans1_b, trans2_w, trans2_b):
    n_blk = _E // _EB
    s_bf, o_bf, conv_out, omega = pl.pallas_call(
        _prep_body,
        grid=(1,),
        in_specs=[
            _full((_N, _D)), _full((_D, _D)), _full((1, _D)),
            _full((_D, _D)), _full((1, _D)), _full((1, _D)),
            _full((_D, _D)), _full((1, _D)), _full((1, _D)), _full((1, _D)),
        ],
        out_specs=[
            _full((_N, _D)), _full((_N, _D)), _full((_N, _D)), _full((_N, _N)),
        ],
        out_shape=[
            jax.ShapeDtypeStruct((_N, _D), _BF16),
            jax.ShapeDtypeStruct((_N, _D), _BF16),
            jax.ShapeDtypeStruct((_N, _D), _F32),
            jax.ShapeDtypeStruct((_N, _N), _F32),
        ],
    )(obj_feats, ws_w, ws_b.reshape(1, _D), wo_w, wo_b.reshape(1, _D),
      w_w.reshape(1, _D), conv_w, conv_b.reshape(1, _D),
      ln1_g.reshape(1, _D), ln1_b.reshape(1, _D))

    src3 = pair_idxs[:, 0].reshape(n_blk, 1, _EB)
    dst3 = pair_idxs[:, 1].reshape(n_blk, 1, _EB)
    atten = pl.pallas_call(
        _edge_body,
        grid=(n_blk,),
        in_specs=[
            pl.BlockSpec((1, 1, _EB), lambda i: (i, 0, 0)),
            pl.BlockSpec((1, 1, _EB), lambda i: (i, 0, 0)),
            pl.BlockSpec((_EB, _D), lambda i: (i, 0)),
            pl.BlockSpec((_N, _D), lambda i: (0, 0)),
            pl.BlockSpec((_N, _D), lambda i: (0, 0)),
            pl.BlockSpec(memory_space=pltpu.SMEM),
        ],
        out_specs=pl.BlockSpec((_N, _N), lambda i: (0, 0)),
        out_shape=jax.ShapeDtypeStruct((_N, _N), _F32),
    )(src3, dst3, phr_feats, s_bf, o_bf, w_b.reshape(1, 1))

    return pl.pallas_call(
        _finish_body,
        grid=(1,),
        in_specs=[
            _full((_N, _N)), _full((_N, _N)), _full((_N, _D)), _full((_N, _D)),
            _full((1, _D)), _full((1, _D)),
            _full((_D, 2 * _D)), _full((1, 2 * _D)),
            _full((2 * _D, _D)), _full((1, _D)),
        ],
        out_specs=_full((_N, _D)),
        out_shape=jax.ShapeDtypeStruct((_N, _D), _F32),
    )(atten, omega, conv_out, obj_feats, ln2_g.reshape(1, _D),
      ln2_b.reshape(1, _D), trans1_w, trans1_b.reshape(1, 2 * _D),
      trans2_w, trans2_b.reshape(1, _D))


# speedup vs baseline: 9.0416x; 1.1129x over previous
import jax
import jax.numpy as jnp
from jax.experimental import pallas as pl
from jax.experimental.pallas import tpu as pltpu

_N = 256
_D = 512
_E = 16384
_EB = 2048
_F32 = jnp.float32
_BF16 = jnp.bfloat16
_HI = jax.lax.Precision.HIGHEST


def _dot3(a, b):
    a_hi = a.astype(_BF16)
    a_lo = (a - a_hi.astype(_F32)).astype(_BF16)
    b_hi = b.astype(_BF16)
    b_lo = (b - b_hi.astype(_F32)).astype(_BF16)
    d = jnp.dot(a_hi, b_hi, preferred_element_type=_F32)
    d += jnp.dot(a_hi, b_lo, preferred_element_type=_F32)
    d += jnp.dot(a_lo, b_hi, preferred_element_type=_F32)
    return d


def _gram3(a):
    dn = (((1,), (1,)), ((), ()))
    a_hi = a.astype(_BF16)
    a_lo = (a - a_hi.astype(_F32)).astype(_BF16)
    g = jax.lax.dot_general(a_hi, a_hi, dn, preferred_element_type=_F32)
    g += jax.lax.dot_general(a_hi, a_lo, dn, preferred_element_type=_F32)
    g += jax.lax.dot_general(a_lo, a_hi, dn, preferred_element_type=_F32)
    return g


def _prep_body(obj_ref, ws_w_ref, ws_b_ref, wo_w_ref, wo_b_ref, w_w_ref,
               conv_w_ref, conv_b_ref, ln1_g_ref, ln1_b_ref,
               s_ref, o_ref, conv_ref, omega_ref):
    obj = obj_ref[...]
    s = jnp.dot(obj, ws_w_ref[...], preferred_element_type=_F32) + ws_b_ref[...]
    s_mod = s * w_w_ref[...]
    o = jnp.dot(obj, wo_w_ref[...], preferred_element_type=_F32) + wo_b_ref[...]
    s_ref[...] = s_mod.astype(_BF16)
    o_ref[...] = o.astype(_BF16)

    g = _gram3(obj)
    rows = jax.lax.broadcasted_iota(jnp.int32, (_N, _N), 0)
    cols = jax.lax.broadcasted_iota(jnp.int32, (_N, _N), 1)
    eye = (rows == cols).astype(_F32)
    diag_col = jnp.sum(g * eye, axis=1, keepdims=True)
    diag_row = jnp.sum(g * eye, axis=0, keepdims=True)
    n2 = jnp.maximum(diag_col + diag_row - 2.0 * g, 0.0)
    omega = jnp.where(n2 < 0.25, 4.0, 0.0)
    omega = jnp.where((n2 >= 0.25) & (n2 < 1.0),
                      1.0 / jnp.maximum(n2, 1e-10), omega)
    omega = jnp.where(rows == cols, 0.0, omega)
    omega_ref[...] = omega

    mu = jnp.mean(obj, axis=1, keepdims=True)
    xc = obj - mu
    var = jnp.mean(xc * xc, axis=1, keepdims=True)
    xn = xc / jnp.sqrt(var + 1e-5) * ln1_g_ref[...] + ln1_b_ref[...]
    conv_ref[...] = jax.nn.relu(
        _dot3(xn, conv_w_ref[...]) + conv_b_ref[...])


def _edge_body(src_ref, dst_ref, phr_ref, s_ref, o_ref, wb_ref, atten_ref):
    src = src_ref[0, 0, :]
    dst = dst_ref[0, 0, :]
    ids = jax.lax.broadcasted_iota(jnp.int32, (_EB, _N), 1)
    oh_s = (src[:, None] == ids).astype(_BF16)
    oh_d = (dst[:, None] == ids).astype(_BF16)
    gs = jnp.dot(oh_s, s_ref[...], preferred_element_type=_F32)
    go = jnp.dot(oh_d, o_ref[...], preferred_element_type=_F32)
    t = gs * go * phr_ref[...]
    af = jnp.sum(t, axis=1) + wb_ref[0, 0]
    weighted = oh_s * af[:, None].astype(_BF16)
    contrib = jax.lax.dot_general(weighted, oh_d, (((0,), (0,)), ((), ())),
                                  preferred_element_type=_F32)

    @pl.when(pl.program_id(0) == 0)
    def _():
        atten_ref[...] = jnp.zeros_like(atten_ref)

    atten_ref[...] += contrib


def _finish_body(atten_ref, omega_ref, conv_ref, obj_ref, ln2_g_ref, ln2_b_ref,
                 t1w_ref, t1b_ref, t2w_ref, t2b_ref, out_ref):
    rows = jax.lax.broadcasted_iota(jnp.int32, (_N, _N), 0)
    cols = jax.lax.broadcasted_iota(jnp.int32, (_N, _N), 1)
    a = atten_ref[...] - jnp.where(rows == cols, 10000.0, 0.0)
    m = jnp.max(a, axis=1, keepdims=True)
    ex = jnp.exp(a - m)
    sm = ex / jnp.sum(ex, axis=1, keepdims=True)
    am = omega_ref[...] * sm
    context = _dot3(am, conv_ref[...])
    outputs = obj_ref[...] + context
    mu = jnp.mean(outputs, axis=1, keepdims=True)
    xc = outputs - mu
    var = jnp.mean(xc * xc, axis=1, keepdims=True)
    xn = xc / jnp.sqrt(var + 1e-5) * ln2_g_ref[...] + ln2_b_ref[...]
    h = jax.nn.relu(_dot3(xn, t1w_ref[...]) + t1b_ref[...])
    trans = _dot3(h, t2w_ref[...]) + t2b_ref[...]
    out_ref[...] = jax.nn.relu(outputs + trans)


def _full(shape, dtype=_F32):
    return pl.BlockSpec(shape, lambda *_: tuple(0 for _ in shape))


def kernel(obj_feats, phr_feats, pair_idxs, ws_w, ws_b, wo_w, wo_b, w_w, w_b,
           conv_w, conv_b, ln1_g, ln1_b, ln2_g, ln2_b,
           trans1_w, trans1_b, trans2_w, trans2_b):
    n_blk = _E // _EB
    s_bf, o_bf, conv_out, omega = pl.pallas_call(
        _prep_body,
        grid=(1,),
        in_specs=[
            _full((_N, _D)), _full((_D, _D)), _full((1, _D)),
            _full((_D, _D)), _full((1, _D)), _full((1, _D)),
            _full((_D, _D)), _full((1, _D)), _full((1, _D)), _full((1, _D)),
        ],
        out_specs=[
            _full((_N, _D)), _full((_N, _D)), _full((_N, _D)), _full((_N, _N)),
        ],
        out_shape=[
            jax.ShapeDtypeStruct((_N, _D), _BF16),
            jax.ShapeDtypeStruct((_N, _D), _BF16),
            jax.ShapeDtypeStruct((_N, _D), _F32),
            jax.ShapeDtypeStruct((_N, _N), _F32),
        ],
    )(obj_feats, ws_w, ws_b.reshape(1, _D), wo_w, wo_b.reshape(1, _D),
      w_w.reshape(1, _D), conv_w, conv_b.reshape(1, _D),
      ln1_g.reshape(1, _D), ln1_b.reshape(1, _D))

    src3 = pair_idxs[:, 0].reshape(n_blk, 1, _EB)
    dst3 = pair_idxs[:, 1].reshape(n_blk, 1, _EB)
    atten = pl.pallas_call(
        _edge_body,
        grid=(n_blk,),
        in_specs=[
            pl.BlockSpec((1, 1, _EB), lambda i: (i, 0, 0)),
            pl.BlockSpec((1, 1, _EB), lambda i: (i, 0, 0)),
            pl.BlockSpec((_EB, _D), lambda i: (i, 0)),
            pl.BlockSpec((_N, _D), lambda i: (0, 0)),
            pl.BlockSpec((_N, _D), lambda i: (0, 0)),
            pl.BlockSpec(memory_space=pltpu.SMEM),
        ],
        out_specs=pl.BlockSpec((_N, _N), lambda i: (0, 0)),
        out_shape=jax.ShapeDtypeStruct((_N, _N), _F32),
    )(src3, dst3, phr_feats, s_bf, o_bf, w_b.reshape(1, 1))

    return pl.pallas_call(
        _finish_body,
        grid=(1,),
        in_specs=[
            _full((_N, _N)), _full((_N, _N)), _full((_N, _D)), _full((_N, _D)),
            _full((1, _D)), _full((1, _D)),
            _full((_D, 2 * _D)), _full((1, 2 * _D)),
            _full((2 * _D, _D)), _full((1, _D)),
        ],
        out_specs=_full((_N, _D)),
        out_shape=jax.ShapeDtypeStruct((_N, _D), _F32),
    )(atten, omega, conv_out, obj_feats, ln2_g.reshape(1, _D),
      ln2_b.reshape(1, _D), trans1_w, trans1_b.reshape(1, 2 * _D),
      trans2_w, trans2_b.reshape(1, _D))


# speedup vs baseline: 10.2309x; 1.1315x over previous
import jax
import jax.numpy as jnp
from jax.experimental import pallas as pl
from jax.experimental.pallas import tpu as pltpu

_N = 256
_D = 512
_E = 16384
_EB = 2048
_F32 = jnp.float32
_BF16 = jnp.bfloat16


def _dot3(a, b):
    a_hi = a.astype(_BF16)
    a_lo = (a - a_hi.astype(_F32)).astype(_BF16)
    b_hi = b.astype(_BF16)
    b_lo = (b - b_hi.astype(_F32)).astype(_BF16)
    d = jnp.dot(a_hi, b_hi, preferred_element_type=_F32)
    d += jnp.dot(a_hi, b_lo, preferred_element_type=_F32)
    d += jnp.dot(a_lo, b_hi, preferred_element_type=_F32)
    return d


def _gram3(a):
    dn = (((1,), (1,)), ((), ()))
    a_hi = a.astype(_BF16)
    a_lo = (a - a_hi.astype(_F32)).astype(_BF16)
    g = jax.lax.dot_general(a_hi, a_hi, dn, preferred_element_type=_F32)
    g += jax.lax.dot_general(a_hi, a_lo, dn, preferred_element_type=_F32)
    g += jax.lax.dot_general(a_lo, a_hi, dn, preferred_element_type=_F32)
    return g


def _body(src_ref, dst_ref, phr_ref, obj_ref, ws_w_ref, ws_b_ref, wo_w_ref,
          wo_b_ref, w_w_ref, conv_w_ref, conv_b_ref, ln1_g_ref, ln1_b_ref,
          wb_ref, ln2_g_ref, ln2_b_ref, t1w_ref, t1b_ref, t2w_ref, t2b_ref,
          out_ref, s_ref, o_ref, conv_ref, omega_ref, atten_ref):
    i = pl.program_id(0)

    @pl.when(i == 0)
    def _prep():
        obj = obj_ref[...]
        s = jnp.dot(obj, ws_w_ref[...], preferred_element_type=_F32)
        s_mod = (s + ws_b_ref[...]) * w_w_ref[...]
        o = jnp.dot(obj, wo_w_ref[...], preferred_element_type=_F32)
        s_ref[...] = s_mod.astype(_BF16)
        o_ref[...] = (o + wo_b_ref[...]).astype(_BF16)

        g = _gram3(obj)
        rows = jax.lax.broadcasted_iota(jnp.int32, (_N, _N), 0)
        cols = jax.lax.broadcasted_iota(jnp.int32, (_N, _N), 1)
        eye = (rows == cols).astype(_F32)
        diag_col = jnp.sum(g * eye, axis=1, keepdims=True)
        diag_row = jnp.sum(g * eye, axis=0, keepdims=True)
        n2 = jnp.maximum(diag_col + diag_row - 2.0 * g, 0.0)
        omega = jnp.where(n2 < 0.25, 4.0, 0.0)
        omega = jnp.where((n2 >= 0.25) & (n2 < 1.0),
                          1.0 / jnp.maximum(n2, 1e-10), omega)
        omega_ref[...] = jnp.where(rows == cols, 0.0, omega)

        mu = jnp.mean(obj, axis=1, keepdims=True)
        xc = obj - mu
        var = jnp.mean(xc * xc, axis=1, keepdims=True)
        xn = xc / jnp.sqrt(var + 1e-5) * ln1_g_ref[...] + ln1_b_ref[...]
        conv_ref[...] = jax.nn.relu(_dot3(xn, conv_w_ref[...])
                                    + conv_b_ref[...])

    @pl.when((i >= 1) & (i <= 8))
    def _edge():
        src = src_ref[0, 0, :]
        dst = dst_ref[0, 0, :]
        ids = jax.lax.broadcasted_iota(jnp.int32, (_EB, _N), 1)
        oh_s = (src[:, None] == ids).astype(_BF16)
        oh_d = (dst[:, None] == ids).astype(_BF16)
        gs = jnp.dot(oh_s, s_ref[...], preferred_element_type=_F32)
        go = jnp.dot(oh_d, o_ref[...], preferred_element_type=_F32)
        t = gs * go * phr_ref[...]
        af = jnp.sum(t, axis=1) + wb_ref[0, 0]
        weighted = oh_s * af[:, None].astype(_BF16)
        contrib = jax.lax.dot_general(weighted, oh_d, (((0,), (0,)), ((), ())),
                                      preferred_element_type=_F32)

        @pl.when(i == 1)
        def _():
            atten_ref[...] = jnp.zeros_like(atten_ref)

        atten_ref[...] += contrib

    @pl.when(i == 9)
    def _finish():
        rows = jax.lax.broadcasted_iota(jnp.int32, (_N, _N), 0)
        cols = jax.lax.broadcasted_iota(jnp.int32, (_N, _N), 1)
        a = atten_ref[...] - jnp.where(rows == cols, 10000.0, 0.0)
        m = jnp.max(a, axis=1, keepdims=True)
        ex = jnp.exp(a - m)
        sm = ex / jnp.sum(ex, axis=1, keepdims=True)
        am = omega_ref[...] * sm
        context = _dot3(am, conv_ref[...])
        outputs = obj_ref[...] + context
        mu = jnp.mean(outputs, axis=1, keepdims=True)
        xc = outputs - mu
        var = jnp.mean(xc * xc, axis=1, keepdims=True)
        xn = xc / jnp.sqrt(var + 1e-5) * ln2_g_ref[...] + ln2_b_ref[...]
        h = jax.nn.relu(_dot3(xn, t1w_ref[...]) + t1b_ref[...])
        trans = _dot3(h, t2w_ref[...]) + t2b_ref[...]
        out_ref[...] = jax.nn.relu(outputs + trans)


def _const(shape):
    return pl.BlockSpec(shape, lambda i: tuple(0 for _ in shape))


def _eblk(i):
    return jnp.maximum(jnp.minimum(i - 1, 7), 0)


def kernel(obj_feats, phr_feats, pair_idxs, ws_w, ws_b, wo_w, wo_b, w_w, w_b,
           conv_w, conv_b, ln1_g, ln1_b, ln2_g, ln2_b,
           trans1_w, trans1_b, trans2_w, trans2_b):
    n_blk = _E // _EB
    src3 = pair_idxs[:, 0].reshape(n_blk, 1, _EB)
    dst3 = pair_idxs[:, 1].reshape(n_blk, 1, _EB)
    return pl.pallas_call(
        _body,
        grid=(n_blk + 2,),
        in_specs=[
            pl.BlockSpec((1, 1, _EB), lambda i: (_eblk(i), 0, 0)),
            pl.BlockSpec((1, 1, _EB), lambda i: (_eblk(i), 0, 0)),
            pl.BlockSpec((_EB, _D), lambda i: (_eblk(i), 0)),
            _const((_N, _D)), _const((_D, _D)), _const((1, _D)),
            _const((_D, _D)), _const((1, _D)), _const((1, _D)),
            _const((_D, _D)), _const((1, _D)), _const((1, _D)),
            _const((1, _D)),
            pl.BlockSpec(memory_space=pltpu.SMEM),
            _const((1, _D)), _const((1, _D)),
            _const((_D, 2 * _D)), _const((1, 2 * _D)),
            _const((2 * _D, _D)), _const((1, _D)),
        ],
        out_specs=_const((_N, _D)),
        out_shape=jax.ShapeDtypeStruct((_N, _D), _F32),
        scratch_shapes=[
            pltpu.VMEM((_N, _D), _BF16),
            pltpu.VMEM((_N, _D), _BF16),
            pltpu.VMEM((_N, _D), _F32),
            pltpu.VMEM((_N, _N), _F32),
            pltpu.VMEM((_N, _N), _F32),
        ],
    )(src3, dst3, phr_feats, obj_feats, ws_w, ws_b.reshape(1, _D),
      wo_w, wo_b.reshape(1, _D), w_w.reshape(1, _D),
      conv_w, conv_b.reshape(1, _D), ln1_g.reshape(1, _D),
      ln1_b.reshape(1, _D), w_b.reshape(1, 1), ln2_g.reshape(1, _D),
      ln2_b.reshape(1, _D), trans1_w, trans1_b.reshape(1, 2 * _D),
      trans2_w, trans2_b.reshape(1, _D))
